# baseline (speedup 1.0000x reference)
import functools
import numpy as np
import jax
import jax.numpy as jnp
from jax import lax
from jax.experimental import pallas as pl
from jax.experimental.pallas import tpu as pltpu
from jax.experimental.pallas import tpu_sc as plsc

_NUM_TIMESTEPS = 1000
_S = 0.008

B, N, M, D = 32, 1000, 8192, 8
BN = B * N

_R = 128
_GRID = BN // _R


def _schedule_tables():
    steps = _NUM_TIMESTEPS + 1
    x = np.linspace(0, _NUM_TIMESTEPS, steps)
    ac = np.cos((x / _NUM_TIMESTEPS + _S) / (1 + _S) * np.pi * 0.5) ** 2
    ac = ac / ac[0]
    betas = np.clip(1 - ac[1:] / ac[:-1], 0.0001, 0.9999)
    acp = np.cumprod(1.0 - betas)
    sqrt_ac = np.sqrt(acp).astype(np.float32)
    sqrt_1mac = np.sqrt(1.0 - acp).astype(np.float32)
    return jnp.asarray(sqrt_ac), jnp.asarray(sqrt_1mac)


def _argmin_body(x_ref, a_ref, x2_ref, a2_ref, idx_ref):
    x_blk = x_ref[...]
    a_blk = a_ref[...]
    cross = jnp.dot(x_blk, a_blk, preferred_element_type=jnp.float32)
    d2 = jnp.maximum(x2_ref[...] - 2.0 * cross + a2_ref[...], 0.0)
    dist = jnp.sqrt(d2)
    mn = jnp.min(dist, axis=1, keepdims=True)
    ii = lax.broadcasted_iota(jnp.int32, (_R, M), 1)
    idx = jnp.min(jnp.where(dist == mn, ii, jnp.int32(M)), axis=1)
    idx_ref[0, 0, :] = idx


def _nn_indices(x2d, a_t, x2, a2):
    out = pl.pallas_call(
        _argmin_body,
        grid=(_GRID,),
        in_specs=[
            pl.BlockSpec((_R, D), lambda i: (i, 0)),
            pl.BlockSpec((D, M), lambda i: (0, 0)),
            pl.BlockSpec((_R, 1), lambda i: (i, 0)),
            pl.BlockSpec((1, M), lambda i: (0, 0)),
        ],
        out_specs=pl.BlockSpec((1, 1, _R), lambda i: (i, 0, 0)),
        out_shape=jax.ShapeDtypeStruct((_GRID, 1, _R), jnp.int32),
    )(x2d, a_t, x2, a2)
    return out.reshape(BN)


def _gather_combine_body(x_hbm, idx_hbm, anc_hbm, sa_hbm, sb_hbm,
                         out_hbm, x_v, idx_v, anc_v, sa_v, sb_v, out_v):
    nc = 2
    w = lax.axis_index("s") * nc + lax.axis_index("c")
    base = w * N
    pltpu.sync_copy(anc_hbm, anc_v)
    pltpu.sync_copy(x_hbm.at[pl.ds(base * D, N * D)], x_v)
    pltpu.sync_copy(idx_hbm.at[pl.ds(base, N)], idx_v.at[pl.ds(0, N)])
    pltpu.sync_copy(sa_hbm.at[pl.ds(w * 16, 16)], sa_v)
    pltpu.sync_copy(sb_hbm.at[pl.ds(w * 16, 16)], sb_v)

    lanes = lax.iota(jnp.int32, 16)
    sa = sa_v[...]
    sb = sb_v[...]
    row_off = lax.shift_right_logical(lanes, 3)
    d_off = lanes & 7

    def body(i, _):
        ridx = plsc.load_gather(idx_v, [i * 2 + row_off])
        addr = ridx * D + d_off
        ag = plsc.load_gather(anc_v, [addr])
        xv = x_v[pl.ds(i * 16, 16)]
        out_v[pl.ds(i * 16, 16)] = sa * xv + sb * ag
        return 0

    lax.fori_loop(0, (N * D) // 16, body, 0)
    pltpu.sync_copy(out_v, out_hbm.at[pl.ds(base * D, N * D)])


def _gather_combine(x_flat, idx, anc_flat, sa_exp, sb_exp):
    mesh = plsc.VectorSubcoreMesh(core_axis_name="c", subcore_axis_name="s")
    f = pl.kernel(
        _gather_combine_body,
        out_type=jax.ShapeDtypeStruct((BN * D,), jnp.float32),
        mesh=mesh,
        compiler_params=pltpu.CompilerParams(needs_layout_passes=False),
        scratch_types=[
            pltpu.VMEM((N * D,), jnp.float32),
            pltpu.VMEM((1024,), jnp.int32),
            pltpu.VMEM((M * D,), jnp.float32),
            pltpu.VMEM((16,), jnp.float32),
            pltpu.VMEM((16,), jnp.float32),
            pltpu.VMEM((N * D,), jnp.float32),
        ],
    )
    return f(x_flat, idx, anc_flat, sa_exp, sb_exp)


def kernel(x_start, t, anchors):
    sqa, sqb = _schedule_tables()
    ti = t.astype(jnp.int32)
    sa_exp = jnp.broadcast_to(sqa[ti][:, None], (B, 16)).reshape(B * 16)
    sb_exp = jnp.broadcast_to(sqb[ti][:, None], (B, 16)).reshape(B * 16)
    x_flat3 = x_start.reshape(B, N, -1)
    a2d = anchors.reshape(M, -1)
    a_t = a2d.T
    x2 = jnp.sum(x_flat3 * x_flat3, axis=-1, keepdims=True)
    a2 = jnp.sum(a2d * a2d, axis=-1)[None, :]
    x2d = x_start.reshape(BN, D)
    idx = _nn_indices(x2d, a_t, x2.reshape(BN, 1), a2)
    out = _gather_combine(
        x2d.reshape(BN * D), idx, anchors.reshape(M * D), sa_exp, sb_exp)
    return out.reshape(B, N, 4, 2)


# speedup vs baseline: 1.0516x; 1.0516x over previous
import numpy as np
import jax
import jax.numpy as jnp
from jax import lax
from jax.experimental import pallas as pl
from jax.experimental.pallas import tpu as pltpu
from jax.experimental.pallas import tpu_sc as plsc

_NUM_TIMESTEPS = 1000
_S = 0.008

B, N, M, D = 32, 1000, 8192, 8
BN = B * N

_R = 256
_GRID = BN // _R


def _schedule_tables():
    steps = _NUM_TIMESTEPS + 1
    x = np.linspace(0, _NUM_TIMESTEPS, steps)
    ac = np.cos((x / _NUM_TIMESTEPS + _S) / (1 + _S) * np.pi * 0.5) ** 2
    ac = ac / ac[0]
    betas = np.clip(1 - ac[1:] / ac[:-1], 0.0001, 0.9999)
    acp = np.cumprod(1.0 - betas)
    sqrt_ac = np.sqrt(acp).astype(np.float32)
    sqrt_1mac = np.sqrt(1.0 - acp).astype(np.float32)
    return jnp.asarray(sqrt_ac), jnp.asarray(sqrt_1mac)


def _argmin_body(x_ref, a_ref, x2_ref, a2_ref, idx_ref):
    x_blk = x_ref[...]
    a_blk = a_ref[...]
    cross = jnp.dot(x_blk, a_blk, preferred_element_type=jnp.float32)
    d2 = jnp.maximum(x2_ref[...] - 2.0 * cross + a2_ref[...], 0.0)
    dist = jnp.sqrt(d2)
    mn = jnp.min(dist, axis=1, keepdims=True)
    ii = lax.broadcasted_iota(jnp.int32, (_R, M), 1)
    idx = jnp.min(jnp.where(dist == mn, ii, jnp.int32(M)), axis=1)
    idx_ref[0, 0, :] = idx


def _nn_indices(x2d, a_t, x2, a2):
    out = pl.pallas_call(
        _argmin_body,
        grid=(_GRID,),
        in_specs=[
            pl.BlockSpec((_R, D), lambda i: (i, 0)),
            pl.BlockSpec((D, M), lambda i: (0, 0)),
            pl.BlockSpec((_R, 1), lambda i: (i, 0)),
            pl.BlockSpec((1, M), lambda i: (0, 0)),
        ],
        out_specs=pl.BlockSpec((1, 1, _R), lambda i: (i, 0, 0)),
        out_shape=jax.ShapeDtypeStruct((_GRID, 1, _R), jnp.int32),
    )(x2d, a_t, x2, a2)
    return out.reshape(BN)


def _gather_combine_body(x_hbm, idx_hbm, anc_hbm, sa_hbm, sb_hbm,
                         out_hbm, x_v, idx_v, anc_v, sa_v, sb_v, out_v):
    nc = 2
    w = lax.axis_index("s") * nc + lax.axis_index("c")
    base = w * N
    pltpu.sync_copy(anc_hbm, anc_v)
    pltpu.sync_copy(x_hbm.at[pl.ds(base * D, N * D)], x_v)
    pltpu.sync_copy(idx_hbm.at[pl.ds(base, N)], idx_v.at[pl.ds(0, N)])
    pltpu.sync_copy(sa_hbm.at[pl.ds(w * 16, 16)], sa_v)
    pltpu.sync_copy(sb_hbm.at[pl.ds(w * 16, 16)], sb_v)

    lanes = lax.iota(jnp.int32, 16)
    sa = sa_v[...]
    sb = sb_v[...]
    row_off = lax.shift_right_logical(lanes, 3)
    d_off = lanes & 7

    def body(i, _):
        ridx = plsc.load_gather(idx_v, [i * 2 + row_off])
        addr = ridx * D + d_off
        ag = plsc.load_gather(anc_v, [addr])
        xv = x_v[pl.ds(i * 16, 16)]
        out_v[pl.ds(i * 16, 16)] = sa * xv + sb * ag
        return 0

    lax.fori_loop(0, (N * D) // 16, body, 0)
    pltpu.sync_copy(out_v, out_hbm.at[pl.ds(base * D, N * D)])


def _gather_combine(x_flat, idx, anc_flat, sa_exp, sb_exp):
    mesh = plsc.VectorSubcoreMesh(core_axis_name="c", subcore_axis_name="s")
    f = pl.kernel(
        _gather_combine_body,
        out_type=jax.ShapeDtypeStruct((BN * D,), jnp.float32),
        mesh=mesh,
        compiler_params=pltpu.CompilerParams(needs_layout_passes=False),
        scratch_types=[
            pltpu.VMEM((N * D,), jnp.float32),
            pltpu.VMEM((1024,), jnp.int32),
            pltpu.VMEM((M * D,), jnp.float32),
            pltpu.VMEM((16,), jnp.float32),
            pltpu.VMEM((16,), jnp.float32),
            pltpu.VMEM((N * D,), jnp.float32),
        ],
    )
    return f(x_flat, idx, anc_flat, sa_exp, sb_exp)


def kernel(x_start, t, anchors):
    sqa, sqb = _schedule_tables()
    ti = t.astype(jnp.int32)
    sa_exp = jnp.broadcast_to(sqa[ti][:, None], (B, 16)).reshape(B * 16)
    sb_exp = jnp.broadcast_to(sqb[ti][:, None], (B, 16)).reshape(B * 16)
    x_flat3 = x_start.reshape(B, N, -1)
    a2d = anchors.reshape(M, -1)
    a_t = a2d.T
    x2 = jnp.sum(x_flat3 * x_flat3, axis=-1, keepdims=True)
    a2 = jnp.sum(a2d * a2d, axis=-1)[None, :]
    x2d = x_start.reshape(BN, D)
    idx = _nn_indices(x2d, a_t, x2.reshape(BN, 1), a2)
    out = _gather_combine(
        x2d.reshape(BN * D), idx, anchors.reshape(M * D), sa_exp, sb_exp)
    return out.reshape(B, N, 4, 2)


# speedup vs baseline: 1.0637x; 1.0115x over previous
import numpy as np
import jax
import jax.numpy as jnp
from jax import lax
from jax.experimental import pallas as pl
from jax.experimental.pallas import tpu as pltpu
from jax.experimental.pallas import tpu_sc as plsc

_NUM_TIMESTEPS = 1000
_S = 0.008

B, N, M, D = 32, 1000, 8192, 8
BN = B * N

_R = 320
_GRID = BN // _R


def _schedule_tables():
    steps = _NUM_TIMESTEPS + 1
    x = np.linspace(0, _NUM_TIMESTEPS, steps)
    ac = np.cos((x / _NUM_TIMESTEPS + _S) / (1 + _S) * np.pi * 0.5) ** 2
    ac = ac / ac[0]
    betas = np.clip(1 - ac[1:] / ac[:-1], 0.0001, 0.9999)
    acp = np.cumprod(1.0 - betas)
    sqrt_ac = np.sqrt(acp).astype(np.float32)
    sqrt_1mac = np.sqrt(1.0 - acp).astype(np.float32)
    return jnp.asarray(sqrt_ac), jnp.asarray(sqrt_1mac)


def _argmin_body(x_ref, a_ref, x2_ref, a2_ref, idx_ref):
    x_blk = x_ref[...]
    a_blk = a_ref[...]
    cross = jnp.dot(x_blk, a_blk, preferred_element_type=jnp.float32)
    d2 = jnp.maximum(x2_ref[...] - 2.0 * cross + a2_ref[...], 0.0)
    dist = jnp.sqrt(d2)
    mn = jnp.min(dist, axis=1, keepdims=True)
    ii = lax.broadcasted_iota(jnp.int32, (_R, M), 1)
    idx = jnp.min(jnp.where(dist == mn, ii, jnp.int32(M)), axis=1)
    idx_ref[0, 0, :] = idx


def _nn_indices(x2d, a_t, x2, a2):
    out = pl.pallas_call(
        _argmin_body,
        grid=(_GRID,),
        in_specs=[
            pl.BlockSpec((_R, D), lambda i: (i, 0)),
            pl.BlockSpec((D, M), lambda i: (0, 0)),
            pl.BlockSpec((_R, 1), lambda i: (i, 0)),
            pl.BlockSpec((1, M), lambda i: (0, 0)),
        ],
        out_specs=pl.BlockSpec((1, 1, _R), lambda i: (i, 0, 0)),
        out_shape=jax.ShapeDtypeStruct((_GRID, 1, _R), jnp.int32),
    )(x2d, a_t, x2, a2)
    return out.reshape(BN)


def _gather_combine_body(x_hbm, idx_hbm, anc_hbm, sa_hbm, sb_hbm,
                         out_hbm, x_v, idx_v, anc_v, sa_v, sb_v, out_v):
    nc = 2
    w = lax.axis_index("s") * nc + lax.axis_index("c")
    base = w * N
    pltpu.sync_copy(anc_hbm, anc_v)
    pltpu.sync_copy(x_hbm.at[pl.ds(base * D, N * D)], x_v)
    pltpu.sync_copy(idx_hbm.at[pl.ds(base, N)], idx_v.at[pl.ds(0, N)])
    pltpu.sync_copy(sa_hbm.at[pl.ds(w * 16, 16)], sa_v)
    pltpu.sync_copy(sb_hbm.at[pl.ds(w * 16, 16)], sb_v)

    lanes = lax.iota(jnp.int32, 16)
    sa = sa_v[...]
    sb = sb_v[...]
    row_off = lax.shift_right_logical(lanes, 3)
    d_off = lanes & 7

    def body(i, _):
        ridx = plsc.load_gather(idx_v, [i * 2 + row_off])
        addr = ridx * D + d_off
        ag = plsc.load_gather(anc_v, [addr])
        xv = x_v[pl.ds(i * 16, 16)]
        out_v[pl.ds(i * 16, 16)] = sa * xv + sb * ag
        return 0

    lax.fori_loop(0, (N * D) // 16, body, 0)
    pltpu.sync_copy(out_v, out_hbm.at[pl.ds(base * D, N * D)])


def _gather_combine(x_flat, idx, anc_flat, sa_exp, sb_exp):
    mesh = plsc.VectorSubcoreMesh(core_axis_name="c", subcore_axis_name="s")
    f = pl.kernel(
        _gather_combine_body,
        out_type=jax.ShapeDtypeStruct((BN * D,), jnp.float32),
        mesh=mesh,
        compiler_params=pltpu.CompilerParams(needs_layout_passes=False),
        scratch_types=[
            pltpu.VMEM((N * D,), jnp.float32),
            pltpu.VMEM((1024,), jnp.int32),
            pltpu.VMEM((M * D,), jnp.float32),
            pltpu.VMEM((16,), jnp.float32),
            pltpu.VMEM((16,), jnp.float32),
            pltpu.VMEM((N * D,), jnp.float32),
        ],
    )
    return f(x_flat, idx, anc_flat, sa_exp, sb_exp)


def kernel(x_start, t, anchors):
    sqa, sqb = _schedule_tables()
    ti = t.astype(jnp.int32)
    sa_exp = jnp.broadcast_to(sqa[ti][:, None], (B, 16)).reshape(B * 16)
    sb_exp = jnp.broadcast_to(sqb[ti][:, None], (B, 16)).reshape(B * 16)
    x_flat3 = x_start.reshape(B, N, -1)
    a2d = anchors.reshape(M, -1)
    a_t = a2d.T
    x2 = jnp.sum(x_flat3 * x_flat3, axis=-1, keepdims=True)
    a2 = jnp.sum(a2d * a2d, axis=-1)[None, :]
    x2d = x_start.reshape(BN, D)
    idx = _nn_indices(x2d, a_t, x2.reshape(BN, 1), a2)
    out = _gather_combine(
        x2d.reshape(BN * D), idx, anchors.reshape(M * D), sa_exp, sb_exp)
    return out.reshape(B, N, 4, 2)
